# baseline (speedup 1.0000x reference)
import jax
import jax.numpy as jnp
from jax.experimental import pallas as pl

_SEQ_BLK = 512


def _body(bin_ref, emb_ref, out_ref, logit_ref):
    amp = bin_ref[...] * 2.0 - 1.0
    table = emb_ref[...]
    out_ref[...] = amp[:, :, None] * table[None, :, :]
    rowsum = jnp.sum(table, axis=1)
    logit_ref[...] = (amp * rowsum[None, :])[:, :, None]


def kernel(binary_input, embeddings):
    seq_len, blen = binary_input.shape
    vocab, emb_sz = embeddings.shape
    grid = (seq_len // _SEQ_BLK,)
    emb, logit = pl.pallas_call(
        _body,
        grid=grid,
        in_specs=[
            pl.BlockSpec((_SEQ_BLK, blen), lambda i: (i, 0)),
            pl.BlockSpec((vocab, emb_sz), lambda i: (0, 0)),
        ],
        out_specs=(
            pl.BlockSpec((_SEQ_BLK, blen, emb_sz), lambda i: (i, 0, 0)),
            pl.BlockSpec((_SEQ_BLK, blen, 1), lambda i: (i, 0, 0)),
        ),
        out_shape=(
            jax.ShapeDtypeStruct((seq_len, blen, emb_sz), jnp.float32),
            jax.ShapeDtypeStruct((seq_len, blen, 1), jnp.float32),
        ),
    )(binary_input, embeddings)
    return emb, logit


# speedup vs baseline: 2.7924x; 2.7924x over previous
import jax
import jax.numpy as jnp
from jax.experimental import pallas as pl

_SEQ_BLK = 512


def _body(bin_ref, emb_ref, out_ref, logit_ref):
    amp = bin_ref[...] * 2.0 - 1.0
    table = emb_ref[...]
    out_ref[...] = amp[:, :, None] * table[None, :, :]
    rowsum = jnp.sum(table, axis=1)
    logit_ref[...] = amp * rowsum[None, :]


def kernel(binary_input, embeddings):
    seq_len, blen = binary_input.shape
    vocab, emb_sz = embeddings.shape
    grid = (seq_len // _SEQ_BLK,)
    emb, logit = pl.pallas_call(
        _body,
        grid=grid,
        in_specs=[
            pl.BlockSpec((_SEQ_BLK, blen), lambda i: (i, 0)),
            pl.BlockSpec((vocab, emb_sz), lambda i: (0, 0)),
        ],
        out_specs=(
            pl.BlockSpec((_SEQ_BLK, blen, emb_sz), lambda i: (i, 0, 0)),
            pl.BlockSpec((_SEQ_BLK, blen), lambda i: (i, 0)),
        ),
        out_shape=(
            jax.ShapeDtypeStruct((seq_len, blen, emb_sz), jnp.float32),
            jax.ShapeDtypeStruct((seq_len, blen), jnp.float32),
        ),
    )(binary_input, embeddings)
    return emb, logit.reshape(seq_len, blen, 1)


# speedup vs baseline: 3.0776x; 1.1021x over previous
import jax
import jax.numpy as jnp
from jax.experimental import pallas as pl

_SEQ_BLK = 512


def _body(binT_ref, emb_ref, out_ref, logitT_ref):
    ampT = binT_ref[...] * 2.0 - 1.0
    table = emb_ref[...]
    out_ref[...] = ampT.T[:, :, None] * table[None, :, :]
    rowsum = jnp.sum(table, axis=1)
    logitT_ref[...] = ampT * rowsum[:, None]


def kernel(binary_input, embeddings):
    seq_len, blen = binary_input.shape
    vocab, emb_sz = embeddings.shape
    grid = (seq_len // _SEQ_BLK,)
    emb, logitT = pl.pallas_call(
        _body,
        grid=grid,
        in_specs=[
            pl.BlockSpec((blen, _SEQ_BLK), lambda i: (0, i)),
            pl.BlockSpec((vocab, emb_sz), lambda i: (0, 0)),
        ],
        out_specs=(
            pl.BlockSpec((_SEQ_BLK, blen, emb_sz), lambda i: (i, 0, 0)),
            pl.BlockSpec((blen, _SEQ_BLK), lambda i: (0, i)),
        ),
        out_shape=(
            jax.ShapeDtypeStruct((seq_len, blen, emb_sz), jnp.float32),
            jax.ShapeDtypeStruct((blen, seq_len), jnp.float32),
        ),
    )(binary_input.T, embeddings)
    return emb, logitT.T.reshape(seq_len, blen, 1)
